# initial kernel scaffold (unmeasured)
import jax
import jax.numpy as jnp
from jax import lax
from jax.experimental import pallas as pl
from jax.experimental.pallas import tpu as pltpu

N_GLOBAL = 4096
EPS = 1e-5


def kernel(x, gamma):
    m, n = x.shape

    def body(x_ref, g_ref, o_ref, partial_ref, recv_ref, send_sem, recv_sem):
        my_x = lax.axis_index("x")
        my_y = lax.axis_index("y")
        nbr = (my_x, 1 - my_y)

        barrier_sem = pltpu.get_barrier_semaphore()
        pl.semaphore_signal(
            barrier_sem, inc=1, device_id=nbr,
            device_id_type=pl.DeviceIdType.MESH,
        )
        pl.semaphore_wait(barrier_sem, 1)

        xv = x_ref[:, :]
        partial_ref[:, :] = jnp.sum(xv * xv, axis=1, keepdims=True)

        rdma = pltpu.make_async_remote_copy(
            src_ref=partial_ref,
            dst_ref=recv_ref,
            send_sem=send_sem,
            recv_sem=recv_sem,
            device_id=nbr,
            device_id_type=pl.DeviceIdType.MESH,
        )
        rdma.start()
        rdma.wait()

        total = partial_ref[:, :] + recv_ref[:, :]
        inv_rms = lax.rsqrt(total / N_GLOBAL + EPS)
        o_ref[:, :] = (xv * inv_rms * g_ref[:, :].reshape(1, n)).astype(
            o_ref.dtype
        )

    return pl.pallas_call(
        body,
        out_shape=jax.ShapeDtypeStruct((m, n), jnp.bfloat16),
        in_specs=[
            pl.BlockSpec(memory_space=pltpu.VMEM),
            pl.BlockSpec(memory_space=pltpu.VMEM),
        ],
        out_specs=pl.BlockSpec(memory_space=pltpu.VMEM),
        scratch_shapes=[
            pltpu.VMEM((m, 1), jnp.float32),
            pltpu.VMEM((m, 1), jnp.float32),
            pltpu.SemaphoreType.DMA,
            pltpu.SemaphoreType.DMA,
        ],
        compiler_params=pltpu.CompilerParams(collective_id=0),
    )(x, gamma)


# baseline (device time: 65848 ns/iter reference)
import jax
import jax.numpy as jnp
from jax import lax
from jax.experimental import pallas as pl
from jax.experimental.pallas import tpu as pltpu

N_GLOBAL = 4096
EPS = 1e-5
M_BLK = 512


def kernel(x, gamma):
    m, n = x.shape
    nblk = m // M_BLK
    pack_rows = M_BLK // 128

    def body(
        x_hbm,
        g_ref,
        o_hbm,
        xb_ref,
        cache_ref,
        ob_ref,
        pack_ref,
        recv_ref,
        invcol_ref,
        in_sems,
        out_sems,
        send_sem,
        recv_sem,
    ):
        my_x = lax.axis_index("x")
        my_y = lax.axis_index("y")
        nbr = (my_x, 1 - my_y)

        barrier_sem = pltpu.get_barrier_semaphore()
        pl.semaphore_signal(
            barrier_sem, inc=1, device_id=nbr,
            device_id_type=pl.DeviceIdType.MESH,
        )
        pl.semaphore_wait(barrier_sem, 1)

        def copy_in(b):
            return pltpu.make_async_copy(
                x_hbm.at[pl.ds(b * M_BLK, M_BLK), :],
                xb_ref.at[b % 2],
                in_sems.at[b % 2],
            )

        copy_in(0).start()
        for b in range(nblk):
            if b + 1 < nblk:
                copy_in(b + 1).start()
            copy_in(b).wait()
            xv = xb_ref[b % 2]
            cache_ref[pl.ds(b * M_BLK, M_BLK), :] = xv.astype(jnp.bfloat16)
            sums = jnp.sum(xv * xv, axis=1)
            pack_ref[pl.ds(b * pack_rows, pack_rows), :] = sums.reshape(
                pack_rows, 128
            )

        rdma = pltpu.make_async_remote_copy(
            src_ref=pack_ref,
            dst_ref=recv_ref,
            send_sem=send_sem,
            recv_sem=recv_sem,
            device_id=nbr,
            device_id_type=pl.DeviceIdType.MESH,
        )
        rdma.start()
        rdma.wait()
        total = pack_ref[:, :] + recv_ref[:, :]
        pack_ref[:, :] = lax.rsqrt(total / N_GLOBAL + EPS)

        eye = jnp.eye(128, dtype=jnp.float32)
        for k in range(m // 128):
            rowv = pack_ref[pl.ds(k, 1), :]
            invcol_ref[pl.ds(k * 128, 128), :] = lax.dot_general(
                eye, rowv, (((1,), (1,)), ((), ()))
            )

        gv = g_ref[:].reshape(1, n).astype(jnp.float32)

        def copy_out(b):
            return pltpu.make_async_copy(
                ob_ref.at[b % 2],
                o_hbm.at[pl.ds(b * M_BLK, M_BLK), :],
                out_sems.at[b % 2],
            )

        for b in range(nblk):
            if b >= 2:
                copy_out(b - 2).wait()
            inv_col = invcol_ref[pl.ds(b * M_BLK, M_BLK), :]
            xc = cache_ref[pl.ds(b * M_BLK, M_BLK), :].astype(jnp.float32)
            ob_ref[b % 2] = (xc * inv_col * gv).astype(jnp.bfloat16)
            copy_out(b).start()
        for b in range(max(nblk - 2, 0), nblk):
            copy_out(b).wait()

    return pl.pallas_call(
        body,
        out_shape=jax.ShapeDtypeStruct((m, n), jnp.bfloat16),
        in_specs=[
            pl.BlockSpec(memory_space=pl.ANY),
            pl.BlockSpec(memory_space=pltpu.VMEM),
        ],
        out_specs=pl.BlockSpec(memory_space=pl.ANY),
        scratch_shapes=[
            pltpu.VMEM((2, M_BLK, n), jnp.float32),
            pltpu.VMEM((m, n), jnp.bfloat16),
            pltpu.VMEM((2, M_BLK, n), jnp.bfloat16),
            pltpu.VMEM((m // 128, 128), jnp.float32),
            pltpu.VMEM((m // 128, 128), jnp.float32),
            pltpu.VMEM((m, 1), jnp.float32),
            pltpu.SemaphoreType.DMA((2,)),
            pltpu.SemaphoreType.DMA((2,)),
            pltpu.SemaphoreType.DMA,
            pltpu.SemaphoreType.DMA,
        ],
        compiler_params=pltpu.CompilerParams(
            collective_id=0, vmem_limit_bytes=64 * 1024 * 1024
        ),
    )(x, gamma)


# device time: 61702 ns/iter; 1.0672x vs baseline; 1.0672x over previous
import jax
import jax.numpy as jnp
from jax import lax
from jax.experimental import pallas as pl
from jax.experimental.pallas import tpu as pltpu

N_GLOBAL = 4096
EPS = 1e-5
M_BLK = 512
LAG = 2


def kernel(x, gamma):
    m, n = x.shape
    nblk = m // M_BLK
    pack_rows = M_BLK // 128

    def body(
        x_hbm,
        g_ref,
        o_hbm,
        xb_ref,
        cache_ref,
        ob_ref,
        pack_ref,
        recv_ref,
        in_sems,
        out_sems,
        send_sems,
        recv_sems,
    ):
        my_x = lax.axis_index("x")
        my_y = lax.axis_index("y")
        nbr = (my_x, 1 - my_y)

        barrier_sem = pltpu.get_barrier_semaphore()
        pl.semaphore_signal(
            barrier_sem, inc=1, device_id=nbr,
            device_id_type=pl.DeviceIdType.MESH,
        )
        pl.semaphore_wait(barrier_sem, 1)

        gv = g_ref[:].reshape(1, n)
        eye = jnp.eye(128, dtype=jnp.float32)

        def copy_in(b):
            return pltpu.make_async_copy(
                x_hbm.at[pl.ds(b * M_BLK, M_BLK), :],
                xb_ref.at[b % 2],
                in_sems.at[b % 2],
            )

        def copy_out(b):
            return pltpu.make_async_copy(
                ob_ref.at[b % 2],
                o_hbm.at[pl.ds(b * M_BLK, M_BLK), :],
                out_sems.at[b % 2],
            )

        def exchange(b):
            sl = pl.ds(b * pack_rows, pack_rows)
            return pltpu.make_async_remote_copy(
                src_ref=pack_ref.at[sl, :],
                dst_ref=recv_ref.at[sl, :],
                send_sem=send_sems.at[b],
                recv_sem=recv_sems.at[b],
                device_id=nbr,
                device_id_type=pl.DeviceIdType.MESH,
            )

        def produce(b):
            if b + 1 < nblk:
                copy_in(b + 1).start()
            copy_in(b).wait()
            xv = xb_ref[b % 2]
            cache_ref[pl.ds(b * M_BLK, M_BLK), :] = (xv * gv).astype(
                jnp.bfloat16
            )
            sums = jnp.sum(xv * xv, axis=1)
            pack_ref[pl.ds(b * pack_rows, pack_rows), :] = sums.reshape(
                pack_rows, 128
            )
            exchange(b).start()

        def consume(b):
            exchange(b).wait_recv()
            sl = pl.ds(b * pack_rows, pack_rows)
            total = pack_ref[sl, :] + recv_ref[sl, :]
            pack_ref[sl, :] = lax.rsqrt(total / N_GLOBAL + EPS)
            if b >= 2:
                copy_out(b - 2).wait()
            for k in range(pack_rows):
                rowv = pack_ref[pl.ds(b * pack_rows + k, 1), :]
                inv_col = lax.dot_general(
                    eye, rowv, (((1,), (1,)), ((), ()))
                )
                xc = cache_ref[
                    pl.ds(b * M_BLK + k * 128, 128), :
                ].astype(jnp.float32)
                ob_ref[b % 2, pl.ds(k * 128, 128), :] = (xc * inv_col).astype(
                    jnp.bfloat16
                )
            copy_out(b).start()

        copy_in(0).start()
        for b in range(nblk):
            produce(b)
            if b >= LAG:
                consume(b - LAG)
        for b in range(nblk - LAG, nblk):
            consume(b)

        for b in range(max(nblk - 2, 0), nblk):
            copy_out(b).wait()
        for b in range(nblk):
            exchange(b).wait_send()

    return pl.pallas_call(
        body,
        out_shape=jax.ShapeDtypeStruct((m, n), jnp.bfloat16),
        in_specs=[
            pl.BlockSpec(memory_space=pl.ANY),
            pl.BlockSpec(memory_space=pltpu.VMEM),
        ],
        out_specs=pl.BlockSpec(memory_space=pl.ANY),
        scratch_shapes=[
            pltpu.VMEM((2, M_BLK, n), jnp.float32),
            pltpu.VMEM((m, n), jnp.bfloat16),
            pltpu.VMEM((2, M_BLK, n), jnp.bfloat16),
            pltpu.VMEM((m // 128, 128), jnp.float32),
            pltpu.VMEM((m // 128, 128), jnp.float32),
            pltpu.SemaphoreType.DMA((2,)),
            pltpu.SemaphoreType.DMA((2,)),
            pltpu.SemaphoreType.DMA((nblk,)),
            pltpu.SemaphoreType.DMA((nblk,)),
        ],
        compiler_params=pltpu.CompilerParams(
            collective_id=0, vmem_limit_bytes=64 * 1024 * 1024
        ),
    )(x, gamma)
